# baseline (speedup 1.0000x reference)
import functools

import jax
import jax.numpy as jnp
from jax import lax
from jax.experimental import pallas as pl
from jax.experimental.pallas import tpu as pltpu
from jax.experimental.pallas import tpu_sc as plsc

_L = 16
_RING = 8
_W = 128


def _mf_kernel(users_hbm, items_hbm, ut_hbm, vt_hbm, utail_hbm, vtail_hbm,
               uet_hbm, iet_hbm, r_hbm,
               uidx_v, iidx_v, warr_v, utail_v, vtail_v,
               outu_v, outv_v, r_v, bufs_and_sems,
               *, bpw, dim, num_cores, nu, nv):
    wid = lax.axis_index("s") * num_cores + lax.axis_index("c")
    base = wid * bpw
    bufs = bufs_and_sems[:_RING]
    sems = bufs_and_sems[_RING:]

    iota = lax.iota(jnp.int32, _L)

    pltpu.sync_copy(users_hbm.at[pl.ds(base, bpw)], uidx_v)
    pltpu.sync_copy(items_hbm.at[pl.ds(base, bpw)], iidx_v)
    pltpu.sync_copy(utail_hbm, utail_v)
    pltpu.sync_copy(vtail_hbm, vtail_v)

    def run_table(tab_hbm, idx_v, tail_v, out_v, tail0, is_v):
        nchunk = bpw // _L

        def wprep(k, carry):
            v = idx_v[pl.ds(k * _L, _L)]
            w = (v >> 7) << 7
            w = jnp.where(v >= tail0, 0, w)
            warr_v[pl.ds(k * _L, _L)] = w
            return carry
        lax.fori_loop(0, nchunk, wprep, 0)
        warr_v[pl.ds(bpw, _L)] = jnp.zeros((_L,), jnp.int32)

        def issue(row_w, slot):
            pltpu.async_copy(
                tab_hbm.at[:, pl.ds(pl.multiple_of(row_w, _W), _W)],
                bufs[slot], sems[slot])

        w0 = warr_v[pl.ds(0, _L)]
        for l in range(_RING):
            issue(w0[l], l)

        def chunk(jj, carry):
            rvec = idx_v[pl.ds(jj * _L, _L)]
            wnext = warr_v[pl.ds(jj * _L + _RING, _L)]
            racc = r_v[pl.ds(jj * _L, _L)]
            for l in range(_L):
                i = jj * _L + l
                slot = l % _RING
                r = rvec[l]
                pltpu.make_async_copy(
                    tab_hbm.at[:, pl.ds(0, _W)], bufs[slot],
                    sems[slot]).wait()
                j = jnp.where(r >= tail0, r - tail0, r & (_W - 1))
                col = jnp.full((_L,), j, jnp.int32)
                g0 = plsc.load_gather(bufs[slot], [iota, col])
                g1 = plsc.load_gather(bufs[slot], [iota + _L, col])
                t0 = plsc.load_gather(tail_v, [iota, col])
                t1 = plsc.load_gather(tail_v, [iota + _L, col])
                tl = r >= tail0
                e0 = jnp.where(tl, t0, g0)
                e1 = jnp.where(tl, t1, g1)
                coli = jnp.full((_L,), i, jnp.int32)
                plsc.store_scatter(out_v, [iota, coli], e0)
                plsc.store_scatter(out_v, [iota + _L, coli], e1)
                if is_v:
                    u0 = plsc.load_gather(outu_v, [iota, coli])
                    u1 = plsc.load_gather(outu_v, [iota + _L, coli])
                    s = jnp.sum(u0 * e0 + u1 * e1)
                    racc = jnp.where(iota == l, s, racc)
                @pl.when(i + _RING < bpw)
                def _():
                    issue(wnext[l], slot)
            if is_v:
                r_v[pl.ds(jj * _L, _L)] = racc
            return carry
        lax.fori_loop(0, nchunk, chunk, 0)

    run_table(ut_hbm, uidx_v, utail_v, outu_v, nu, False)
    run_table(vt_hbm, iidx_v, vtail_v, outv_v, nv, True)

    pltpu.sync_copy(outu_v, uet_hbm.at[:, pl.ds(base, bpw)])
    pltpu.sync_copy(outv_v, iet_hbm.at[:, pl.ds(base, bpw)])
    pltpu.sync_copy(r_v, r_hbm.at[pl.ds(base, bpw)])


def kernel(users, items, U, V):
    batch = users.shape[0]
    dim = U.shape[1]
    nrow_u = U.shape[0]
    nrow_v = V.shape[0]
    users = users.astype(jnp.int32)
    items = items.astype(jnp.int32)

    ut = U.T
    vt = V.T
    nu = (nrow_u // _W) * _W
    nv = (nrow_v // _W) * _W
    utail = jnp.zeros((dim, _W), jnp.float32).at[:, :nrow_u - nu].set(
        U[nu:].T)
    vtail = jnp.zeros((dim, _W), jnp.float32).at[:, :nrow_v - nv].set(
        V[nv:].T)

    info = plsc.get_sparse_core_info()
    num_workers = info.num_cores * info.num_subcores
    bpw = batch // num_workers

    mesh = plsc.VectorSubcoreMesh(core_axis_name="c", subcore_axis_name="s")

    scratch = [
        pltpu.VMEM((bpw,), jnp.int32),
        pltpu.VMEM((bpw,), jnp.int32),
        pltpu.VMEM((bpw + 2 * _RING,), jnp.int32),
        pltpu.VMEM((dim, _W), jnp.float32),
        pltpu.VMEM((dim, _W), jnp.float32),
        pltpu.VMEM((dim, bpw), jnp.float32),
        pltpu.VMEM((dim, bpw), jnp.float32),
        pltpu.VMEM((bpw,), jnp.float32),
    ]
    scratch += [pltpu.VMEM((dim, _W), jnp.float32) for _ in range(_RING)]
    scratch += [pltpu.SemaphoreType.DMA for _ in range(_RING)]

    def body(users_h, items_h, ut_h, vt_h, utail_h, vtail_h,
             uet_h, iet_h, r_h, uidx, iidx, warr, utl, vtl,
             outu, outv, rr, *ring):
        _mf_kernel(users_h, items_h, ut_h, vt_h, utail_h, vtail_h,
                   uet_h, iet_h, r_h, uidx, iidx, warr, utl, vtl,
                   outu, outv, rr, list(ring),
                   bpw=bpw, dim=dim, num_cores=info.num_cores,
                   nu=nu, nv=nv)

    mf = pl.kernel(
        body,
        out_type=(
            jax.ShapeDtypeStruct((dim, batch), jnp.float32),
            jax.ShapeDtypeStruct((dim, batch), jnp.float32),
            jax.ShapeDtypeStruct((batch,), jnp.float32),
        ),
        mesh=mesh,
        compiler_params=pltpu.CompilerParams(needs_layout_passes=False,
                                             use_tc_tiling_on_sc=True),
        scratch_types=scratch,
    )

    uet, iet, r_hats = mf(users, items, ut, vt, utail, vtail)
    return (uet.T, iet.T, r_hats)
